# Initial kernel scaffold; baseline (speedup 1.0000x reference)
import jax
import jax.numpy as jnp
from jax.experimental import pallas as pl


def kernel(x, edge_index, W1, b1, Wo, bo):
    raise NotImplementedError("write your pallas kernel here")


# speedup vs baseline: 16.0378x; 16.0378x over previous
import functools

import jax
import jax.numpy as jnp
from jax import lax
from jax.experimental import pallas as pl
from jax.experimental.pallas import tpu as pltpu
from jax.experimental.pallas import tpu_sc as plsc

N = 10000
E = 320000
C_IN = 128
C_HID = 128
C_OUT = 64
HEADS = 4
CW = 64

NC = 2
NS = 16
NW = NC * NS

B = 128
JMAX = 79
EPT = JMAX * B
E_PAD = NW * EPT
ACC_ROWS = 10240
ROWS_PER_TILE_Z = ACC_ROWS // NS
FLUSH_SMALL = 624
FLUSH_LAST = N - (NS - 1) * FLUSH_SMALL
DUMP = N


@functools.cache
def _mesh():
    return plsc.VectorSubcoreMesh(
        core_axis_name="c", subcore_axis_name="s", num_cores=NC, num_subcores=NS
    )


_SC_PARAMS = pltpu.CompilerParams(use_tc_tiling_on_sc=False)


def _zero_buf(buf):
    rows, cols = buf.shape

    @pl.loop(0, rows)
    def _(i):
        @pl.loop(0, cols, step=16)
        def _(k):
            buf[i, pl.ds(k, 16)] = jnp.zeros((16,), jnp.float32)


def _fill_ones(buf):
    rows, cols = buf.shape

    @pl.loop(0, rows)
    def _(i):
        @pl.loop(0, cols, step=16)
        def _(k):
            buf[i, pl.ds(k, 16)] = jnp.ones((16,), jnp.float32)


def _zero_acc(acc, zbuf, sid):
    nz = ROWS_PER_TILE_Z // B

    @pl.loop(0, nz)
    def _(r):
        pltpu.sync_copy(zbuf, acc.at[pl.ds(sid * ROWS_PER_TILE_Z + r * B, B)])


def _flush(acc, out_hbm, cid, sid):
    start = pl.multiple_of(sid * FLUSH_SMALL, 8)

    @pl.when(sid < NS - 1)
    def _():
        pltpu.sync_copy(
            acc.at[pl.ds(start, FLUSH_SMALL)],
            out_hbm.at[cid].at[pl.ds(start, FLUSH_SMALL)],
        )

    @pl.when(sid == NS - 1)
    def _():
        base = (NS - 1) * FLUSH_SMALL
        pltpu.sync_copy(
            acc.at[pl.ds(base, FLUSH_LAST)],
            out_hbm.at[cid].at[pl.ds(base, FLUSH_LAST)],
        )


@functools.cache
def _make_deg():
    return functools.partial(
        pl.kernel,
        out_type=jax.ShapeDtypeStruct((NC, N, 16), jnp.float32),
        mesh=_mesh(),
        scratch_types=[
            pltpu.VMEM((JMAX, B), jnp.int32),
            pltpu.VMEM((B, 16), jnp.float32),
            pltpu.VMEM_SHARED((ACC_ROWS, 16), jnp.float32),
        ],
        compiler_params=_SC_PARAMS,
    )(_deg_body)


def _deg_body(dst_hbm, out_hbm, dst_v, ones_v, acc):
    cid = lax.axis_index("c")
    sid = lax.axis_index("s")
    wid = cid * NS + sid

    pltpu.sync_copy(dst_hbm.at[wid], dst_v)

    _zero_buf(ones_v)
    _zero_acc(acc, ones_v, sid)
    plsc.subcore_barrier()

    _fill_ones(ones_v)

    @pl.loop(0, JMAX)
    def _(j):
        pltpu.sync_copy(ones_v, acc.at[dst_v.at[j]], add=True)

    plsc.subcore_barrier()
    _flush(acc, out_hbm, cid, sid)


@functools.cache
def _make_spmm(n_tab):
    @functools.partial(
        pl.kernel,
        out_type=[jax.ShapeDtypeStruct((NC, N, CW), jnp.float32)] * n_tab,
        mesh=_mesh(),
        scratch_types=[
            pltpu.VMEM((JMAX, B), jnp.int32),
            pltpu.VMEM((JMAX, B), jnp.int32),
            pltpu.VMEM((B, CW), jnp.float32),
            pltpu.VMEM((B, CW), jnp.float32),
            pltpu.VMEM_SHARED((ACC_ROWS, CW), jnp.float32),
            pltpu.SemaphoreType.DMA,
            pltpu.SemaphoreType.DMA,
        ],
        compiler_params=_SC_PARAMS,
    )
    def spmm(src_hbm, dst_hbm, *rest):
        tabs = rest[:n_tab]
        outs = rest[n_tab : 2 * n_tab]
        src_v, dst_v, rows_a, rows_b, acc, sem_a, sem_b = rest[2 * n_tab :]

        cid = lax.axis_index("c")
        sid = lax.axis_index("s")
        wid = cid * NS + sid

        pltpu.sync_copy(src_hbm.at[wid], src_v)
        pltpu.sync_copy(dst_hbm.at[wid], dst_v)

        for t in range(n_tab):
            tab = tabs[t]

            _zero_buf(rows_a)
            _zero_acc(acc, rows_a, sid)
            plsc.subcore_barrier()

            pltpu.async_copy(tab.at[src_v.at[0]], rows_a, sem_a)

            @pl.loop(0, JMAX)
            def _(j):
                even = (j % 2) == 0
                nxt = j + 1

                @pl.when(jnp.logical_and(even, nxt < JMAX))
                def _():
                    pltpu.async_copy(tab.at[src_v.at[nxt]], rows_b, sem_b)

                @pl.when(jnp.logical_and(~even, nxt < JMAX))
                def _():
                    pltpu.async_copy(tab.at[src_v.at[nxt]], rows_a, sem_a)

                @pl.when(even)
                def _():
                    pltpu.make_async_copy(
                        tab.at[src_v.at[j]], rows_a, sem_a
                    ).wait()
                    pltpu.sync_copy(rows_a, acc.at[dst_v.at[j]], add=True)

                @pl.when(~even)
                def _():
                    pltpu.make_async_copy(
                        tab.at[src_v.at[j]], rows_b, sem_b
                    ).wait()
                    pltpu.sync_copy(rows_b, acc.at[dst_v.at[j]], add=True)

            plsc.subcore_barrier()
            _flush(acc, outs[t], cid, sid)
            plsc.subcore_barrier()

    return spmm


def _dinv_from_degp(degp):
    deg = degp[0] + degp[1] + 1.0
    return lax.rsqrt(deg[:, 0:1])


def _tc1_body(x_ref, w1_ref, degp_ref, xs0_ref, xs1_ref):
    dinv = _dinv_from_degp(degp_ref[...])
    xw = jnp.dot(x_ref[...], w1_ref[...], preferred_element_type=jnp.float32)
    xs = xw * dinv
    xs0_ref[...] = xs[:, :CW]
    xs1_ref[...] = xs[:, CW:]


def _tc2_body(
    s10_ref, s11_ref, xs0_ref, xs1_ref, degp_ref, b1_ref, wof_ref, *hs_refs
):
    dinv = _dinv_from_degp(degp_ref[...])
    h0 = jnp.maximum(
        (s10_ref[0] + s10_ref[1] + xs0_ref[...]) * dinv + b1_ref[:, :CW], 0.0
    )
    h1 = jnp.maximum(
        (s11_ref[0] + s11_ref[1] + xs1_ref[...]) * dinv + b1_ref[:, CW:], 0.0
    )
    hs = jnp.dot(h0, wof_ref[:CW, :], preferred_element_type=jnp.float32)
    hs = hs + jnp.dot(h1, wof_ref[CW:, :], preferred_element_type=jnp.float32)
    hs = hs * dinv
    for k, hs_ref in enumerate(hs_refs):
        hs_ref[...] = hs[:, k * CW : (k + 1) * CW]


def _tc3_body(*refs):
    s2_refs = refs[:4]
    hs_refs = refs[4:8]
    degp_ref, bof_ref, o_ref = refs[8:]
    dinv = _dinv_from_degp(degp_ref[...])
    for k in range(4):
        ok = (s2_refs[k][0] + s2_refs[k][1] + hs_refs[k][...]) * dinv
        o_ref[:, k * CW : (k + 1) * CW] = ok + bof_ref[:, k * CW : (k + 1) * CW]


def _row_block(shape, rb, row_axis):
    blk = list(shape)
    blk[row_axis] = rb
    nd = len(shape)

    def idx(i):
        return tuple(i if d == row_axis else 0 for d in range(nd))

    return pl.BlockSpec(tuple(blk), idx)


def kernel(x, edge_index, W1, b1, Wo, bo):
    src = edge_index[0].astype(jnp.int32)
    dst = edge_index[1].astype(jnp.int32)

    pad = E_PAD - E
    src_t = jnp.concatenate([src, jnp.zeros((pad,), jnp.int32)]).reshape(
        NW, JMAX, B
    )
    dst_t = jnp.concatenate(
        [dst, jnp.full((pad,), DUMP, jnp.int32)]
    ).reshape(NW, JMAX, B)

    wof = Wo.transpose(1, 0, 2).reshape(C_HID, HEADS * C_OUT)
    bof = bo.reshape(1, HEADS * C_OUT)
    b1r = b1.reshape(1, C_HID)

    degp = _make_deg()(dst_t)

    rb = 2000
    grid = (N // rb,)
    f32 = jnp.float32

    xs0, xs1 = pl.pallas_call(
        _tc1_body,
        grid=grid,
        in_specs=[
            _row_block((N, C_IN), rb, 0),
            pl.BlockSpec((C_IN, C_HID), lambda i: (0, 0)),
            _row_block((NC, N, 16), rb, 1),
        ],
        out_specs=[_row_block((N, CW), rb, 0)] * 2,
        out_shape=[jax.ShapeDtypeStruct((N, CW), f32)] * 2,
    )(x, W1, degp)

    s10, s11 = _make_spmm(2)(src_t, dst_t, xs0, xs1)

    hs = pl.pallas_call(
        _tc2_body,
        grid=grid,
        in_specs=[
            _row_block((NC, N, CW), rb, 1),
            _row_block((NC, N, CW), rb, 1),
            _row_block((N, CW), rb, 0),
            _row_block((N, CW), rb, 0),
            _row_block((NC, N, 16), rb, 1),
            pl.BlockSpec((1, C_HID), lambda i: (0, 0)),
            pl.BlockSpec((C_HID, HEADS * C_OUT), lambda i: (0, 0)),
        ],
        out_specs=[_row_block((N, CW), rb, 0)] * 4,
        out_shape=[jax.ShapeDtypeStruct((N, CW), f32)] * 4,
    )(s10, s11, xs0, xs1, degp, b1r, wof)

    s2 = _make_spmm(4)(src_t, dst_t, *hs)

    out_flat = pl.pallas_call(
        _tc3_body,
        grid=grid,
        in_specs=[_row_block((NC, N, CW), rb, 1)] * 4
        + [_row_block((N, CW), rb, 0)] * 4
        + [
            _row_block((NC, N, 16), rb, 1),
            pl.BlockSpec((1, HEADS * C_OUT), lambda i: (0, 0)),
        ],
        out_specs=_row_block((N, HEADS * C_OUT), rb, 0),
        out_shape=jax.ShapeDtypeStruct((N, HEADS * C_OUT), f32),
    )(*s2, *hs, degp, bof)

    return out_flat.reshape(N, HEADS, C_OUT).transpose(1, 0, 2)
